# baseline (speedup 1.0000x reference)
import functools

import jax
import jax.numpy as jnp
from jax import lax
from jax.experimental import pallas as pl
from jax.experimental.pallas import tpu as pltpu
from jax.experimental.pallas import tpu_sc as plsc

_Q = 1024
_D = 64
_KB = 2000


def _argmin_body(q_ref, a_ref, idx_out_ref, best_d_ref, best_i_ref):
    i = pl.program_id(0)
    q = q_ref[...]
    a = a_ref[...]
    q_sq = jnp.sum(q * q, axis=1, keepdims=True)
    k_sq = jnp.sum(a * a, axis=1)
    dot = lax.dot_general(q, a, (((1,), (1,)), ((), ())),
                          preferred_element_type=jnp.float32)
    d2 = q_sq - 2.0 * dot + k_sq[None, :]
    dist = jnp.sqrt(jnp.maximum(d2, 0.0))
    bmin = jnp.min(dist, axis=1, keepdims=True)
    col = lax.broadcasted_iota(jnp.int32, dist.shape, 1)
    bidx = jnp.min(jnp.where(dist == bmin, col, _KB), axis=1, keepdims=True)
    bidx = bidx + i * _KB

    @pl.when(i == 0)
    def _():
        best_d_ref[...] = bmin
        best_i_ref[...] = bidx

    @pl.when(i > 0)
    def _():
        upd = bmin < best_d_ref[...]
        best_d_ref[...] = jnp.where(upd, bmin, best_d_ref[...])
        best_i_ref[...] = jnp.where(upd, bidx, best_i_ref[...])

    @pl.when(i == pl.num_programs(0) - 1)
    def _():
        idx_out_ref[...] = best_i_ref[...]


def _tc_argmin(pred_action, action_set):
    k = action_set.shape[0]
    return pl.pallas_call(
        _argmin_body,
        grid=(k // _KB,),
        in_specs=[
            pl.BlockSpec((_Q, _D), lambda i: (0, 0)),
            pl.BlockSpec((_KB, _D), lambda i: (i, 0)),
        ],
        out_specs=pl.BlockSpec((_Q, 1), lambda i: (0, 0)),
        out_shape=jax.ShapeDtypeStruct((_Q, 1), jnp.int32),
        scratch_shapes=[
            pltpu.VMEM((_Q, 1), jnp.float32),
            pltpu.VMEM((_Q, 1), jnp.int32),
        ],
    )(pred_action, action_set)


def _sc_gather(action_set, idx):
    info = plsc.get_sparse_core_info()
    nw = info.num_cores * info.num_subcores
    bpw = _Q // nw
    nc = info.num_cores
    mesh = plsc.VectorSubcoreMesh(core_axis_name="c", subcore_axis_name="s")

    @functools.partial(
        pl.kernel,
        mesh=mesh,
        out_type=jax.ShapeDtypeStruct((_Q, _D), jnp.float32),
        compiler_params=pltpu.CompilerParams(use_tc_tiling_on_sc=False),
        scratch_types=[
            pltpu.VMEM((bpw,), jnp.int32),
            pltpu.VMEM((bpw, _D), jnp.float32),
            pltpu.SemaphoreType.DMA,
        ],
    )
    def gather(table_hbm, idx_hbm, out_hbm, idx_v, rows_v, sem):
        wid = lax.axis_index("s") * nc + lax.axis_index("c")
        base = wid * bpw
        pltpu.sync_copy(idx_hbm.at[pl.ds(base, bpw)], idx_v)
        pltpu.async_copy(table_hbm.at[idx_v], rows_v, sem).wait()
        pltpu.sync_copy(rows_v, out_hbm.at[pl.ds(base, bpw)])

    return gather(action_set, idx)


def kernel(pred_action, action_set):
    idx = _tc_argmin(pred_action, action_set).reshape(_Q)
    return _sc_gather(action_set, idx)


# speedup vs baseline: 1.2871x; 1.2871x over previous
import functools

import jax
import jax.numpy as jnp
from jax import lax
from jax.experimental import pallas as pl
from jax.experimental.pallas import tpu as pltpu
from jax.experimental.pallas import tpu_sc as plsc

_Q = 1024
_D = 64
_KB = 2000


def _argmin_body(q_ref, a_ref, idx_out_ref, best_d_ref, best_i_ref):
    i = pl.program_id(0)
    q = q_ref[...]
    a = a_ref[...]
    q_sq = jnp.sum(q * q, axis=1, keepdims=True)
    k_sq = jnp.sum(a * a, axis=1)
    dot = lax.dot_general(q, a, (((1,), (1,)), ((), ())),
                          preferred_element_type=jnp.float32)
    d2 = q_sq - 2.0 * dot + k_sq[None, :]
    m2 = jnp.maximum(jnp.min(d2, axis=1, keepdims=True), 0.0)
    bmin = jnp.sqrt(m2)
    thr = m2
    x = m2
    for _ in range(3):
        x = lax.bitcast_convert_type(
            lax.bitcast_convert_type(x, jnp.int32) + 1, jnp.float32)
        thr = jnp.where(jnp.sqrt(x) == bmin, x, thr)
    col = lax.broadcasted_iota(jnp.int32, d2.shape, 1)
    bidx = jnp.min(jnp.where(d2 <= thr, col, _KB), axis=1, keepdims=True)
    bidx = bidx + i * _KB

    @pl.when(i == 0)
    def _():
        best_d_ref[...] = bmin
        best_i_ref[...] = bidx

    @pl.when(i > 0)
    def _():
        upd = bmin < best_d_ref[...]
        best_d_ref[...] = jnp.where(upd, bmin, best_d_ref[...])
        best_i_ref[...] = jnp.where(upd, bidx, best_i_ref[...])

    @pl.when(i == pl.num_programs(0) - 1)
    def _():
        idx_out_ref[...] = best_i_ref[...]


def _tc_argmin(pred_action, action_set):
    k = action_set.shape[0]
    return pl.pallas_call(
        _argmin_body,
        grid=(k // _KB,),
        in_specs=[
            pl.BlockSpec((_Q, _D), lambda i: (0, 0)),
            pl.BlockSpec((_KB, _D), lambda i: (i, 0)),
        ],
        out_specs=pl.BlockSpec((_Q, 1), lambda i: (0, 0)),
        out_shape=jax.ShapeDtypeStruct((_Q, 1), jnp.int32),
        scratch_shapes=[
            pltpu.VMEM((_Q, 1), jnp.float32),
            pltpu.VMEM((_Q, 1), jnp.int32),
        ],
    )(pred_action, action_set)


def _sc_gather(action_set, idx):
    info = plsc.get_sparse_core_info()
    nw = info.num_cores * info.num_subcores
    bpw = _Q // nw
    nc = info.num_cores
    mesh = plsc.VectorSubcoreMesh(core_axis_name="c", subcore_axis_name="s")

    @functools.partial(
        pl.kernel,
        mesh=mesh,
        out_type=jax.ShapeDtypeStruct((_Q, _D), jnp.float32),
        compiler_params=pltpu.CompilerParams(use_tc_tiling_on_sc=False),
        scratch_types=[
            pltpu.VMEM((bpw,), jnp.int32),
            pltpu.VMEM((bpw, _D), jnp.float32),
            pltpu.SemaphoreType.DMA,
        ],
    )
    def gather(table_hbm, idx_hbm, out_hbm, idx_v, rows_v, sem):
        wid = lax.axis_index("s") * nc + lax.axis_index("c")
        base = wid * bpw
        pltpu.sync_copy(idx_hbm.at[pl.ds(base, bpw)], idx_v)
        pltpu.async_copy(table_hbm.at[idx_v], rows_v, sem).wait()
        pltpu.sync_copy(rows_v, out_hbm.at[pl.ds(base, bpw)])

    return gather(action_set, idx)


def kernel(pred_action, action_set):
    idx = _tc_argmin(pred_action, action_set).reshape(_Q)
    return _sc_gather(action_set, idx)


# speedup vs baseline: 1.3490x; 1.0481x over previous
import functools

import jax
import jax.numpy as jnp
from jax import lax
from jax.experimental import pallas as pl
from jax.experimental.pallas import tpu as pltpu
from jax.experimental.pallas import tpu_sc as plsc

_Q = 1024
_D = 64
_KB = 2000


def _argmin_body(q2_ref, qsq_ref, a_ref, idx_out_ref, best_d_ref, best_i_ref):
    i = pl.program_id(0)
    q2 = q2_ref[...]
    q_sq = qsq_ref[...]
    a = a_ref[...]
    k_sq = jnp.sum(a * a, axis=1)
    dot2 = lax.dot_general(q2, a, (((1,), (1,)), ((), ())),
                           preferred_element_type=jnp.float32)
    d2 = (q_sq - dot2) + k_sq[None, :]
    m2 = jnp.maximum(jnp.min(d2, axis=1, keepdims=True), 0.0)
    bmin = jnp.sqrt(m2)
    thr = m2
    x = m2
    for _ in range(3):
        x = lax.bitcast_convert_type(
            lax.bitcast_convert_type(x, jnp.int32) + 1, jnp.float32)
        thr = jnp.where(jnp.sqrt(x) == bmin, x, thr)
    col = lax.broadcasted_iota(jnp.int32, d2.shape, 1)
    bidx = jnp.min(jnp.where(d2 <= thr, col, _KB), axis=1, keepdims=True)
    bidx = bidx + i * _KB

    @pl.when(i == 0)
    def _():
        best_d_ref[...] = bmin
        best_i_ref[...] = bidx

    @pl.when(i > 0)
    def _():
        upd = bmin < best_d_ref[...]
        best_d_ref[...] = jnp.where(upd, bmin, best_d_ref[...])
        best_i_ref[...] = jnp.where(upd, bidx, best_i_ref[...])

    @pl.when(i == pl.num_programs(0) - 1)
    def _():
        idx_out_ref[...] = best_i_ref[...]


def _tc_argmin(pred_action, action_set):
    k = action_set.shape[0]
    q2 = pred_action + pred_action
    q_sq = jnp.sum(pred_action * pred_action, axis=1, keepdims=True)
    return pl.pallas_call(
        _argmin_body,
        grid=(k // _KB,),
        in_specs=[
            pl.BlockSpec((_Q, _D), lambda i: (0, 0)),
            pl.BlockSpec((_Q, 1), lambda i: (0, 0)),
            pl.BlockSpec((_KB, _D), lambda i: (i, 0)),
        ],
        out_specs=pl.BlockSpec((_Q, 1), lambda i: (0, 0)),
        out_shape=jax.ShapeDtypeStruct((_Q, 1), jnp.int32),
        scratch_shapes=[
            pltpu.VMEM((_Q, 1), jnp.float32),
            pltpu.VMEM((_Q, 1), jnp.int32),
        ],
    )(q2, q_sq, action_set)


def _sc_gather(table128, idx):
    k = table128.shape[0]
    info = plsc.get_sparse_core_info()
    nw = info.num_cores * info.num_subcores
    bpw = _Q // nw
    nc = info.num_cores
    mesh = plsc.VectorSubcoreMesh(core_axis_name="c", subcore_axis_name="s")

    @functools.partial(
        pl.kernel,
        mesh=mesh,
        out_type=jax.ShapeDtypeStruct((_Q, 128), jnp.float32),
        scratch_types=[
            pltpu.VMEM((bpw,), jnp.int32),
            pltpu.VMEM((bpw, 128), jnp.float32),
            pltpu.SemaphoreType.DMA,
        ],
    )
    def gather(table_hbm, idx_hbm, out_hbm, idx_v, rows_v, sem):
        wid = lax.axis_index("s") * nc + lax.axis_index("c")
        base = wid * bpw
        pltpu.sync_copy(idx_hbm.at[pl.ds(base, bpw)], idx_v)
        pltpu.async_copy(table_hbm.at[idx_v], rows_v, sem).wait()
        pltpu.sync_copy(rows_v, out_hbm.at[pl.ds(base, bpw)])

    return gather(table128, idx)


def kernel(pred_action, action_set):
    idx = _tc_argmin(pred_action, action_set).reshape(_Q)
    table128 = jnp.pad(action_set, ((0, 0), (0, 128 - _D)))
    return _sc_gather(table128, idx)[:, :_D]
